# Initial kernel scaffold; baseline (speedup 1.0000x reference)
import jax
import jax.numpy as jnp
from jax.experimental import pallas as pl


def kernel(all_rois, gt_boxes, num_boxes, gt_3d_info):
    raise NotImplementedError("write your pallas kernel here")


# speedup vs baseline: 4.8572x; 4.8572x over previous
import functools

import jax
import jax.numpy as jnp
from jax.experimental import pallas as pl
from jax.experimental.pallas import tpu as pltpu

_N = 20000
_G = 20
_NR = _N + _G
_ROWS = 160
_NP = _ROWS * 128
_K = 128
_KFG = 32
_NCLS_STD = (0.1, 0.1, 0.2, 0.2)


def _body(coords_ref, gt_ref, nb_ref, info_ref, out_ref):
    x1 = coords_ref[0, 0]
    y1 = coords_ref[0, 1]
    x2 = coords_ref[0, 2]
    y2 = coords_ref[0, 3]
    area = (x2 - x1 + 1.0) * (y2 - y1 + 1.0)

    nb = nb_ref[0, 0, 0]
    run_max = jnp.full((_ROWS, 128), -2.0, jnp.float32)
    best_g = jnp.zeros((_ROWS, 128), jnp.float32)
    for g in range(_G):
        gx1 = gt_ref[0, g, 0]
        gy1 = gt_ref[0, g, 1]
        gx2 = gt_ref[0, g, 2]
        gy2 = gt_ref[0, g, 3]
        iw = jnp.clip(jnp.minimum(x2, gx2) - jnp.maximum(x1, gx1) + 1.0, 0.0)
        ih = jnp.clip(jnp.minimum(y2, gy2) - jnp.maximum(y1, gy1) + 1.0, 0.0)
        inter = iw * ih
        garea = (gx2 - gx1 + 1.0) * (gy2 - gy1 + 1.0)
        iou = inter / (area + garea - inter + 1e-6)
        val = jnp.where(g < nb, iou, -1.0)
        upd = val > run_max
        run_max = jnp.where(upd, val, run_max)
        best_g = jnp.where(upd, jnp.float32(g), best_g)

    fg = run_max >= 0.5
    bg = jnp.logical_and(run_max < 0.5, run_max >= 0.1)
    priority = run_max + jnp.where(fg, 10.0, 0.0) + jnp.where(bg, 5.0, 0.0)

    gidx = (jax.lax.broadcasted_iota(jnp.int32, (_ROWS, 128), 0) * 128
            + jax.lax.broadcasted_iota(jnp.int32, (_ROWS, 128), 1)
            ).astype(jnp.float32)
    priority = jnp.where(gidx < float(_NR), priority, -1.0)

    lane = jax.lax.broadcasted_iota(jnp.int32, (1, 128), 1).astype(jnp.float32)

    def step(i, carry):
        prio, sx1, sy1, sx2, sy2, sbg, sm = carry
        m = jnp.max(prio)
        idx = jnp.min(jnp.where(prio == m, gidx, 1e9))
        sel = gidx == idx
        prio = jnp.where(sel, -3.0, prio)
        selm = jnp.where(sel, 1.0, 0.0)
        onehot = jnp.where(lane == i.astype(jnp.float32), 1.0, 0.0)
        sx1 = sx1 + jnp.sum(selm * x1) * onehot
        sy1 = sy1 + jnp.sum(selm * y1) * onehot
        sx2 = sx2 + jnp.sum(selm * x2) * onehot
        sy2 = sy2 + jnp.sum(selm * y2) * onehot
        sbg = sbg + jnp.sum(selm * best_g) * onehot
        sm = sm + m * onehot
        return prio, sx1, sy1, sx2, sy2, sbg, sm

    zero_row = jnp.zeros((1, 128), jnp.float32)
    prio, sx1, sy1, sx2, sy2, sbg, sm = jax.lax.fori_loop(
        0, _K, step,
        (priority, zero_row, zero_row, zero_row, zero_row, zero_row, zero_row))

    fg_row = sm >= 8.0
    sel_fg = jnp.logical_and(fg_row, lane < float(_KFG))

    lab = zero_row
    gx1r = zero_row
    gy1r = zero_row
    gx2r = zero_row
    gy2r = zero_row
    for g in range(_G):
        hit = sbg == jnp.float32(g)
        lab = jnp.where(hit, gt_ref[0, g, 4], lab)
        gx1r = jnp.where(hit, gt_ref[0, g, 0], gx1r)
        gy1r = jnp.where(hit, gt_ref[0, g, 1], gy1r)
        gx2r = jnp.where(hit, gt_ref[0, g, 2], gx2r)
        gy2r = jnp.where(hit, gt_ref[0, g, 3], gy2r)
    labels = jnp.where(sel_fg, lab, 0.0)

    ew = jnp.maximum(sx2 - sx1 + 1.0, 1e-6)
    eh = jnp.maximum(sy2 - sy1 + 1.0, 1e-6)
    ecx = sx1 + 0.5 * ew
    ecy = sy1 + 0.5 * eh
    gw = jnp.maximum(gx2r - gx1r + 1.0, 1e-6)
    gh = jnp.maximum(gy2r - gy1r + 1.0, 1e-6)
    gcx = gx1r + 0.5 * gw
    gcy = gy1r + 0.5 * gh
    dx = (gcx - ecx) / ew / _NCLS_STD[0]
    dy = (gcy - ecy) / eh / _NCLS_STD[1]
    dw = jnp.log(gw / ew) / _NCLS_STD[2]
    dh = jnp.log(gh / eh) / _NCLS_STD[3]
    fgf = jnp.where(sel_fg, 1.0, 0.0)
    dx = dx * fgf
    dy = dy * fgf
    dw = dw * fgf
    dh = dh * fgf

    infos = []
    for d in range(7):
        acc = zero_row
        for g in range(_G):
            acc = jnp.where(sbg == jnp.float32(g), info_ref[0, g, d], acc)
        infos.append(acc)

    rows = [sx1, sy1, sx2, sy2, labels, fgf, dx, dy, dw, dh,
            gx1r, gy1r, gx2r, gy2r] + infos + [zero_row, zero_row, zero_row]
    out_ref[0] = jnp.concatenate(rows, axis=0)


@jax.jit
def kernel(all_rois, gt_boxes, num_boxes, gt_3d_info):
    B = all_rois.shape[0]
    coords = jnp.concatenate([all_rois[:, :, 1:5], gt_boxes[:, :, :4]], axis=1)
    coords = jnp.pad(coords, ((0, 0), (0, _NP - _NR), (0, 0)))
    coords = coords.transpose(0, 2, 1).reshape(B, 4, _ROWS, 128)

    planes = pl.pallas_call(
        _body,
        grid=(B,),
        in_specs=[
            pl.BlockSpec((1, 4, _ROWS, 128), lambda b: (b, 0, 0, 0)),
            pl.BlockSpec((1, _G, 5), lambda b: (b, 0, 0),
                         memory_space=pltpu.SMEM),
            pl.BlockSpec((1, 1, 1), lambda b: (b, 0, 0),
                         memory_space=pltpu.SMEM),
            pl.BlockSpec((1, _G, 7), lambda b: (b, 0, 0),
                         memory_space=pltpu.SMEM),
        ],
        out_specs=pl.BlockSpec((1, 24, 128), lambda b: (b, 0, 0)),
        out_shape=jax.ShapeDtypeStruct((B, 24, 128), jnp.float32),
    )(coords, gt_boxes, num_boxes.astype(jnp.int32).reshape(B, 1, 1),
      gt_3d_info)

    sx1 = planes[:, 0]
    sy1 = planes[:, 1]
    sx2 = planes[:, 2]
    sy2 = planes[:, 3]
    labels = planes[:, 4]
    fgf = planes[:, 5]
    rois = jnp.stack([jnp.zeros_like(sx1), sx1, sy1, sx2, sy2], axis=-1)
    bbox_targets = planes[:, 6:10].transpose(0, 2, 1)
    inside_w = jnp.broadcast_to(fgf[:, :, None], (B, _K, 4))
    outside_w = inside_w
    rois_for_3d = rois[:, :_KFG]
    gt_bbox_for_3d = planes[:, 10:14].transpose(0, 2, 1)[:, :_KFG]
    gt_3d_info_rois = planes[:, 14:21].transpose(0, 2, 1)[:, :_KFG]
    return (rois, labels, bbox_targets, inside_w, outside_w,
            rois_for_3d, gt_bbox_for_3d, gt_3d_info_rois)
